# Initial kernel scaffold; baseline (speedup 1.0000x reference)
import jax
import jax.numpy as jnp
from jax.experimental import pallas as pl


def kernel(x, edge_index, W1s, W1n, b1, W2s, W2n, b2, W3s, W3n, b3, Wp1, bp1, Wp2, bp2):
    raise NotImplementedError("write your pallas kernel here")


# speedup vs baseline: 4.3616x; 4.3616x over previous
import functools

import jax
import jax.numpy as jnp
from jax import lax
from jax.experimental import pallas as pl
from jax.experimental.pallas import tpu as pltpu
from jax.experimental.pallas import tpu_sc as plsc

_N = 10000
_E = 320000
_D_IN = 128
_H = 64

_NUM_TILES = 32
_SUBCORES = 16
_K = 128
_E_PAD = 327680
_EPT = _E_PAD // _NUM_TILES
_NBLK = _EPT // _K
_N_PAD = 10112
_RPT = _N_PAD // _SUBCORES


def _sc_agg_kernel(want_deg):
    mesh = plsc.VectorSubcoreMesh(core_axis_name="c", subcore_axis_name="s")
    out_type = [
        jax.ShapeDtypeStruct((_N_PAD, _H), jnp.float32),
        jax.ShapeDtypeStruct((_N_PAD, _H), jnp.float32),
    ]
    scratch = [
        pltpu.VMEM((_K,), jnp.int32),
        pltpu.VMEM((_K,), jnp.int32),
        pltpu.VMEM((_K, _H), jnp.float32),
        pltpu.VMEM((_RPT, _H), jnp.float32),
        pltpu.VMEM_SHARED((_N_PAD, _H), jnp.float32),
        pltpu.SemaphoreType.DMA,
    ]
    if want_deg:
        out_type += [
            jax.ShapeDtypeStruct((_N_PAD, 16), jnp.float32),
            jax.ShapeDtypeStruct((_N_PAD, 16), jnp.float32),
        ]
        scratch += [
            pltpu.VMEM((_K, 16), jnp.float32),
            pltpu.VMEM((_RPT, 16), jnp.float32),
            pltpu.VMEM_SHARED((_N_PAD, 16), jnp.float32),
        ]

    def body(g_hbm, src_hbm, dst_hbm, z64_hbm, z16_hbm, *refs):
        if want_deg:
            (out0, out1, deg0, deg1, src_v, dst_v, rows_v, stage_v, acc_sh,
             sem, ones_v, dstage_v, dacc_sh) = refs
        else:
            out0, out1, src_v, dst_v, rows_v, stage_v, acc_sh, sem = refs
        c = lax.axis_index("c")
        s = lax.axis_index("s")
        wid = c * _SUBCORES + s
        r0 = s * _RPT

        pltpu.sync_copy(z64_hbm.at[pl.ds(r0, _RPT)], stage_v)
        pltpu.sync_copy(stage_v, acc_sh.at[pl.ds(r0, _RPT)])
        if want_deg:
            pltpu.sync_copy(z16_hbm.at[pl.ds(r0, _RPT)], dstage_v)
            pltpu.sync_copy(dstage_v, dacc_sh.at[pl.ds(r0, _RPT)])
            for i in range(_K):
                ones_v[i, :] = jnp.ones((16,), jnp.float32)
        plsc.subcore_barrier()

        ebase = wid * _EPT

        def blk(i, carry):
            off = ebase + i * _K
            pltpu.sync_copy(src_hbm.at[pl.ds(off, _K)], src_v)
            pltpu.sync_copy(dst_hbm.at[pl.ds(off, _K)], dst_v)
            pltpu.async_copy(g_hbm.at[src_v], rows_v, sem).wait()
            pltpu.sync_copy(rows_v, acc_sh.at[dst_v], add=True)
            if want_deg:
                pltpu.sync_copy(ones_v, dacc_sh.at[dst_v], add=True)
            return carry

        lax.fori_loop(0, _NBLK, blk, 0)
        plsc.subcore_barrier()

        pltpu.sync_copy(acc_sh.at[pl.ds(r0, _RPT)], stage_v)

        @pl.when(c == 0)
        def _():
            pltpu.sync_copy(stage_v, out0.at[pl.ds(r0, _RPT)])

        @pl.when(c == 1)
        def _():
            pltpu.sync_copy(stage_v, out1.at[pl.ds(r0, _RPT)])

        if want_deg:
            pltpu.sync_copy(dacc_sh.at[pl.ds(r0, _RPT)], dstage_v)

            @pl.when(c == 0)
            def _():
                pltpu.sync_copy(dstage_v, deg0.at[pl.ds(r0, _RPT)])

            @pl.when(c == 1)
            def _():
                pltpu.sync_copy(dstage_v, deg1.at[pl.ds(r0, _RPT)])

    return pl.kernel(body, out_type=out_type, mesh=mesh, scratch_types=scratch,
                     compiler_params=pltpu.CompilerParams(
                         use_tc_tiling_on_sc=False),
                     name="sc_edge_agg_deg" if want_deg else "sc_edge_agg")


_sc_agg_deg = _sc_agg_kernel(True)
_sc_agg = _sc_agg_kernel(False)


def _tc_proj_body(x_ref, wn_ref, ws_ref, g_ref, hs_ref):
    x = x_ref[...]
    g_ref[...] = jnp.dot(x, wn_ref[...], preferred_element_type=jnp.float32)
    hs_ref[...] = jnp.dot(x, ws_ref[...], preferred_element_type=jnp.float32)


def _tc_proj(x, wn, ws):
    return pl.pallas_call(
        _tc_proj_body,
        out_shape=[
            jax.ShapeDtypeStruct((_N_PAD, _H), jnp.float32),
            jax.ShapeDtypeStruct((_N_PAD, _H), jnp.float32),
        ],
    )(x, wn, ws)


def _combine(hs, s0, s1, deg0, deg1, b):
    deg = deg0[:, :1] + deg1[:, :1]
    inv = 1.0 / jnp.maximum(deg, 1.0)
    h = hs + (s0 + s1) * inv + b
    rows = lax.broadcasted_iota(jnp.int32, (_N_PAD, 1), 0)
    return h, rows < _N


def _tc_mid_body(hs_ref, s0_ref, s1_ref, d0_ref, d1_ref, b_ref,
                 wn_ref, ws_ref, g_ref, hs2_ref, m_ref):
    h, valid = _combine(hs_ref[...], s0_ref[...], s1_ref[...],
                        d0_ref[...], d1_ref[...], b_ref[...])
    h = jnp.where(valid, jnp.maximum(h, 0.0), 0.0)
    m_ref[...] = jnp.max(jnp.where(valid, h, -jnp.inf), axis=0, keepdims=True)
    g_ref[...] = jnp.dot(h, wn_ref[...], preferred_element_type=jnp.float32)
    hs2_ref[...] = jnp.dot(h, ws_ref[...], preferred_element_type=jnp.float32)


def _tc_mid(hs, s0, s1, d0, d1, b, wn, ws):
    return pl.pallas_call(
        _tc_mid_body,
        out_shape=[
            jax.ShapeDtypeStruct((_N_PAD, _H), jnp.float32),
            jax.ShapeDtypeStruct((_N_PAD, _H), jnp.float32),
            jax.ShapeDtypeStruct((1, _H), jnp.float32),
        ],
    )(hs, s0, s1, d0, d1, b, wn, ws)


def _tc_final_body(hs_ref, s0_ref, s1_ref, d0_ref, d1_ref, b_ref,
                   m1_ref, m2_ref, wp1_ref, bp1_ref, wp2_ref, bp2_ref, y_ref):
    h, valid = _combine(hs_ref[...], s0_ref[...], s1_ref[...],
                        d0_ref[...], d1_ref[...], b_ref[...])
    m3 = jnp.max(jnp.where(valid, h, -jnp.inf), axis=0, keepdims=True)
    mm = jnp.concatenate([m1_ref[...], m2_ref[...], m3], axis=1)
    hid = jnp.dot(mm, wp1_ref[...], preferred_element_type=jnp.float32)
    hid = hid + bp1_ref[...]
    y = jnp.dot(hid, wp2_ref[...], preferred_element_type=jnp.float32)
    y_ref[...] = y + bp2_ref[...]


def _tc_final(hs, s0, s1, d0, d1, b, m1, m2, wp1, bp1, wp2, bp2):
    return pl.pallas_call(
        _tc_final_body,
        out_shape=jax.ShapeDtypeStruct((1, 10), jnp.float32),
    )(hs, s0, s1, d0, d1, b, m1, m2, wp1, bp1, wp2, bp2)


def kernel(x, edge_index, W1s, W1n, b1, W2s, W2n, b2, W3s, W3n, b3,
           Wp1, bp1, Wp2, bp2):
    xp = jnp.zeros((_N_PAD, _D_IN), jnp.float32).at[:_N].set(x)
    pad = jnp.full((_E_PAD - _E,), _N, jnp.int32)
    src = jnp.concatenate([edge_index[0], pad])
    dst = jnp.concatenate([edge_index[1], pad])
    z64 = jnp.zeros((_N_PAD, _H), jnp.float32)
    z16 = jnp.zeros((_N_PAD, 16), jnp.float32)
    b1r = b1.reshape(1, _H)
    b2r = b2.reshape(1, _H)
    b3r = b3.reshape(1, _H)

    g1, hs1 = _tc_proj(xp, W1n, W1s)
    s10, s11, d0, d1 = _sc_agg_deg(g1, src, dst, z64, z16)
    g2, hs2, m1 = _tc_mid(hs1, s10, s11, d0, d1, b1r, W2n, W2s)
    s20, s21 = _sc_agg(g2, src, dst, z64, z16)
    g3, hs3, m2 = _tc_mid(hs2, s20, s21, d0, d1, b2r, W3n, W3s)
    s30, s31 = _sc_agg(g3, src, dst, z64, z16)
    return _tc_final(hs3, s30, s31, d0, d1, b3r, m1, m2,
                     Wp1, bp1.reshape(1, -1), Wp2, bp2.reshape(1, -1))
